# baseline (speedup 1.0000x reference)
import functools

import jax
import jax.numpy as jnp
from jax import lax
from jax.experimental import pallas as pl
from jax.experimental.pallas import tpu as pltpu
from jax.experimental.pallas import tpu_sc as plsc

NUM_ROWS_TABLE = 1_000_000
EMBED_DIM = 32
BATCH_SIZE = 16384

NC = 2
NS = 16
LANES = 16
NW = NC * NS
B_PER_W = BATCH_SIZE // NW
IDX_CHUNK = 128
CHUNKS = B_PER_W // IDX_CHUNK
UNROLL = 16


def _body(uid_hbm, iid_hbm, uemb_hbm, iemb_hbm, ub_hbm, ib_hbm, mean_hbm,
          out_hbm, uidx_v, iidx_v, u_buf0, u_buf1, i_buf0, i_buf1,
          ub_v, ib_v, out_v, mean_v, emb_sem0, emb_sem1, bias_sem):
    wid = lax.axis_index("s") * NC + lax.axis_index("c")
    base_row = wid * CHUNKS
    base = wid * B_PER_W

    pltpu.sync_copy(uid_hbm.at[pl.ds(base_row, CHUNKS)], uidx_v)
    pltpu.sync_copy(iid_hbm.at[pl.ds(base_row, CHUNKS)], iidx_v)
    pltpu.sync_copy(mean_hbm, mean_v)

    bias_copies = []
    for j in range(CHUNKS):
        sl = pl.ds(j * IDX_CHUNK, IDX_CHUNK)
        bias_copies.append(pltpu.async_copy(
            ub_hbm.at[uidx_v.at[j]], ub_v.at[sl], bias_sem))
        bias_copies.append(pltpu.async_copy(
            ib_hbm.at[iidx_v.at[j]], ib_v.at[sl], bias_sem))

    u_bufs = (u_buf0, u_buf1)
    i_bufs = (i_buf0, i_buf1)
    sems = (emb_sem0, emb_sem1)

    def fire(j):
        b = j & 1
        return (pltpu.async_copy(uemb_hbm.at[uidx_v.at[j]], u_bufs[b], sems[b]),
                pltpu.async_copy(iemb_hbm.at[iidx_v.at[j]], i_bufs[b], sems[b]))

    inflight = [fire(0), fire(1)]

    H = EMBED_DIM // 2
    lane = lax.iota(jnp.int32, LANES)
    zeros16 = jnp.zeros((LANES,), jnp.float32)
    for j in range(CHUNKS):
        b = j & 1
        for c in inflight[j]:
            c.wait()
        u_buf, i_buf = u_bufs[b], i_bufs[b]
        out_base = j * IDX_CHUNK

        def step(s, carry, u_buf=u_buf, i_buf=i_buf, out_base=out_base):
            r0 = s * LANES
            acc = zeros16
            for r in range(LANES):
                u0 = u_buf[r0 + r, pl.ds(0, H)]
                u1 = u_buf[r0 + r, pl.ds(H, H)]
                i0 = i_buf[r0 + r, pl.ds(0, H)]
                i1 = i_buf[r0 + r, pl.ds(H, H)]
                p = u0 * i0 + u1 * i1
                acc = jnp.where(lane == r, jnp.sum(p), acc)
            out_v[pl.ds(out_base + r0, LANES)] = acc
            return carry

        lax.fori_loop(0, IDX_CHUNK // LANES, step, 0)

        if j + 2 < CHUNKS:
            inflight.append(fire(j + 2))

    for c in bias_copies:
        c.wait()

    mean16 = mean_v[...]
    for g in range(B_PER_W // LANES):
        sl = pl.ds(g * LANES, LANES)
        out_v[sl] = out_v[sl] + ub_v[sl] + ib_v[sl] + mean16

    pltpu.sync_copy(out_v, out_hbm.at[pl.ds(base, B_PER_W)])


@jax.jit
def _run(u_id2d, i_id2d, user_emb, item_emb, ub_flat, ib_flat, mean16):
    mesh = plsc.VectorSubcoreMesh(core_axis_name="c", subcore_axis_name="s")
    call = pl.kernel(
        _body,
        out_type=jax.ShapeDtypeStruct((BATCH_SIZE,), jnp.float32),
        mesh=mesh,
        compiler_params=pltpu.CompilerParams(
            needs_layout_passes=False, use_tc_tiling_on_sc=False),
        scratch_types=[
            pltpu.VMEM((CHUNKS, IDX_CHUNK), jnp.int32),
            pltpu.VMEM((CHUNKS, IDX_CHUNK), jnp.int32),
            pltpu.VMEM((IDX_CHUNK, EMBED_DIM), jnp.float32),
            pltpu.VMEM((IDX_CHUNK, EMBED_DIM), jnp.float32),
            pltpu.VMEM((IDX_CHUNK, EMBED_DIM), jnp.float32),
            pltpu.VMEM((IDX_CHUNK, EMBED_DIM), jnp.float32),
            pltpu.VMEM((B_PER_W,), jnp.float32),
            pltpu.VMEM((B_PER_W,), jnp.float32),
            pltpu.VMEM((B_PER_W,), jnp.float32),
            pltpu.VMEM((LANES,), jnp.float32),
            pltpu.SemaphoreType.DMA,
            pltpu.SemaphoreType.DMA,
            pltpu.SemaphoreType.DMA,
        ],
    )
    return call(u_id2d, i_id2d, user_emb, item_emb, ub_flat, ib_flat, mean16)


def kernel(u_id, i_id, user_emb, item_emb, user_bias, item_bias, mean):
    u_id2d = u_id.astype(jnp.int32).reshape(NW * CHUNKS, IDX_CHUNK)
    i_id2d = i_id.astype(jnp.int32).reshape(NW * CHUNKS, IDX_CHUNK)
    ub_flat = user_bias.reshape(-1)
    ib_flat = item_bias.reshape(-1)
    mean16 = jnp.broadcast_to(mean.astype(jnp.float32).reshape(()), (LANES,))
    return _run(u_id2d, i_id2d, user_emb, item_emb, ub_flat, ib_flat, mean16)
